# baseline (speedup 1.0000x reference)
import jax
import jax.numpy as jnp
from jax.experimental import pallas as pl
from jax.experimental.pallas import tpu as pltpu

_GRID = 16.0
_NUM_NEG = 16
_SOS_NEG = 8
_MARGIN = 1.0
_BIG = 1e30


def _nearest4(x, y):
    n = x.shape[0]
    jx = jnp.clip(jnp.floor(x * (1.0 / _GRID)), 0.0, 31.0)
    jy = jnp.clip(jnp.floor(y * (1.0 / _GRID)), 0.0, 31.0)
    c0 = jnp.clip(jx - 2.0, 0.0, 27.0)
    r0 = jnp.clip(jy - 2.0, 0.0, 27.0)
    lane = jax.lax.broadcasted_iota(jnp.int32, (n, 25), 1).astype(jnp.float32)
    dcol = lane - 5.0 * jnp.floor(lane * 0.2)
    drow = jnp.floor(lane * 0.2)
    cols = c0 + dcol
    rows = r0 + drow
    cx = cols * _GRID + 8.0
    cy = rows * _GRID + 8.0
    dx = x - cx
    dy = y - cy
    d2 = dx * dx + dy * dy
    idx = rows * 32.0 + cols
    ids = []
    for _ in range(4):
        m = jnp.min(d2, axis=1, keepdims=True)
        sel = jnp.min(jnp.where(d2 == m, idx, jnp.float32(4096.0)),
                      axis=1, keepdims=True)
        ids.append(sel.astype(jnp.int32))
        d2 = jnp.where(idx == sel, jnp.float32(_BIG), d2)
    return ids


def _onehot4(ids, lane_hw):
    m = (lane_hw == ids[0]) | (lane_hw == ids[1]) \
        | (lane_hw == ids[2]) | (lane_hw == ids[3])
    return m.astype(jnp.float32)


def _loss_kernel(homo_ref, kp1_ref, wkp1_ref, kd_ref, d_ref, out_ref):
    i = pl.program_id(0)
    n = kd_ref.shape[1]
    hw = d_ref.shape[2]

    kd = kd_ref[0]
    D = d_ref[0]

    kx = kp1_ref[0, :, 0:1]
    ky = kp1_ref[0, :, 1:2]
    wx = wkp1_ref[0, :, 0:1]
    wy = wkp1_ref[0, :, 1:2]

    lane_hw = jax.lax.broadcasted_iota(jnp.int32, (n, hw), 1)

    kids = _nearest4(kx, ky)
    wids = _nearest4(wx, wy)
    K = _onehot4(kids, lane_hw)
    W = _onehot4(wids, lane_hw)

    h00 = homo_ref[i, 0]
    h01 = homo_ref[i, 1]
    h02 = homo_ref[i, 2]
    h10 = homo_ref[i, 3]
    h11 = homo_ref[i, 4]
    h12 = homo_ref[i, 5]
    h20 = homo_ref[i, 6]
    h21 = homo_ref[i, 7]
    h22 = homo_ref[i, 8]
    N = None
    for p in range(4):
        cidf = kids[p].astype(jnp.float32)
        row = jnp.floor(cidf * (1.0 / 32.0))
        col = cidf - 32.0 * row
        cx = col * _GRID + 8.0
        cy = row * _GRID + 8.0
        wz = h20 * cx + h21 * cy + h22
        px = (h00 * cx + h01 * cy + h02) / (wz + 1e-8)
        py = (h10 * cx + h11 * cy + h12) / (wz + 1e-8)
        cnt = _onehot4(_nearest4(px, py), lane_hw)
        N = cnt if N is None else N + cnt

    bx = wx * (1.0 / _GRID) - 0.5
    by = wy * (1.0 / _GRID) - 0.5
    x0 = jnp.floor(bx)
    y0 = jnp.floor(by)
    fx = bx - x0
    fy = by - y0
    x0c = jnp.clip(x0, 0.0, 31.0)
    x1c = jnp.clip(x0 + 1.0, 0.0, 31.0)
    y0c = jnp.clip(y0, 0.0, 31.0)
    y1c = jnp.clip(y0 + 1.0, 0.0, 31.0)
    i00 = (y0c * 32.0 + x0c).astype(jnp.int32)
    i01 = (y0c * 32.0 + x1c).astype(jnp.int32)
    i10 = (y1c * 32.0 + x0c).astype(jnp.int32)
    i11 = (y1c * 32.0 + x1c).astype(jnp.int32)
    z = jnp.float32(0.0)
    A = (jnp.where(lane_hw == i00, (1 - fy) * (1 - fx), z)
         + jnp.where(lane_hw == i01, (1 - fy) * fx, z)
         + jnp.where(lane_hw == i10, fy * (1 - fx), z)
         + jnp.where(lane_hw == i11, fy * fx, z))
    nt = (((1,), (1,)), ((), ()))
    wd = jax.lax.dot_general(A, D, nt, preferred_element_type=jnp.float32)
    wd = wd * jax.lax.rsqrt(jnp.sum(wd * wd, axis=1, keepdims=True) + 1e-12)

    pos = 2.0 - 2.0 * jnp.sum(kd * wd, axis=1, keepdims=True)

    S = jax.lax.dot_general(kd, D, (((1,), (0,)), ((), ())),
                            preferred_element_type=jnp.float32)
    X = 2.0 - 2.0 * S + 5.0 * N
    fos_sum = jnp.float32(0.0)
    rem = jnp.full((n, 1), float(_NUM_NEG), jnp.float32)
    pm = pos + _MARGIN
    for _ in range(_NUM_NEG):
        m = jnp.min(X, axis=1, keepdims=True)
        eq = (X == m).astype(jnp.float32)
        cnt = jnp.sum(eq, axis=1, keepdims=True)
        take = jnp.minimum(cnt, rem)
        rem = rem - take
        X = X + eq * _BIG
        t = jnp.maximum(pm - m, 0.0)
        fos_sum = fos_sum + jnp.sum(take * t * t)

    Km = jax.lax.dot_general(K, K, nt, preferred_element_type=jnp.float32)
    Wm = jax.lax.dot_general(N, W, nt, preferred_element_type=jnp.float32)
    kraw = 2.0 - 2.0 * jax.lax.dot_general(kd, kd, nt,
                                           preferred_element_type=jnp.float32)
    wraw = 2.0 - 2.0 * jax.lax.dot_general(wd, wd, nt,
                                           preferred_element_type=jnp.float32)
    Xa = kraw + 5.0 * Km
    Xb = wraw + 5.0 * Wm
    lane_n = jax.lax.broadcasted_iota(jnp.int32, (n, n), 1).astype(jnp.float32)
    nf = jnp.float32(n)
    sacc = jnp.zeros((n, 1), jnp.float32)
    for _ in range(_SOS_NEG):
        ma = jnp.min(Xa, axis=1, keepdims=True)
        sa = jnp.min(jnp.where(Xa == ma, lane_n, nf), axis=1, keepdims=True)
        ea = lane_n == sa
        va = jnp.sum(jnp.where(ea, kraw, 0.0), axis=1, keepdims=True)
        Xa = jnp.where(ea, jnp.float32(_BIG), Xa)
        mb = jnp.min(Xb, axis=1, keepdims=True)
        sb = jnp.min(jnp.where(Xb == mb, lane_n, nf), axis=1, keepdims=True)
        eb = lane_n == sb
        vb = jnp.sum(jnp.where(eb, wraw, 0.0), axis=1, keepdims=True)
        Xb = jnp.where(eb, jnp.float32(_BIG), Xb)
        d = va - vb
        sacc = sacc + d * d
    sos_sum = jnp.sum(jnp.sqrt(sacc + 1e-12))

    lane_o = jax.lax.broadcasted_iota(jnp.int32, (1, 128), 1)
    out_ref[0] = jnp.where(lane_o == 0, fos_sum,
                           jnp.where(lane_o == 1, sos_sum, 0.0))


@jax.jit
def kernel(kp1, w_kp1, kp1_desc, desc2, homo12):
    b, n, c = kp1_desc.shape
    h, w = desc2.shape[2], desc2.shape[3]
    hw = h * w
    D = desc2.reshape(b, c, hw)
    homo_flat = homo12.reshape(b, 9)

    grid_spec = pltpu.PrefetchScalarGridSpec(
        num_scalar_prefetch=1,
        grid=(b,),
        in_specs=[
            pl.BlockSpec((1, n, 2), lambda i, s: (i, 0, 0)),
            pl.BlockSpec((1, n, 2), lambda i, s: (i, 0, 0)),
            pl.BlockSpec((1, n, c), lambda i, s: (i, 0, 0)),
            pl.BlockSpec((1, c, hw), lambda i, s: (i, 0, 0)),
        ],
        out_specs=pl.BlockSpec((1, 1, 128), lambda i, s: (i, 0, 0)),
    )
    part = pl.pallas_call(
        _loss_kernel,
        grid_spec=grid_spec,
        out_shape=jax.ShapeDtypeStruct((b, 1, 128), jnp.float32),
    )(homo_flat, kp1, w_kp1, kp1_desc, D)
    fos = jnp.sum(part[:, 0, 0]) / (b * n * _NUM_NEG)
    sos = jnp.sum(part[:, 0, 1]) / (b * n)
    return fos + sos


# speedup vs baseline: 1.0686x; 1.0686x over previous
import jax
import jax.numpy as jnp
from jax.experimental import pallas as pl
from jax.experimental.pallas import tpu as pltpu

_GRID = 16.0
_NUM_NEG = 16
_SOS_NEG = 8
_MARGIN = 1.0
_BIG = 1e30


def _nearest4(x, y):
    n = x.shape[0]
    jx = jnp.clip(jnp.floor(x * (1.0 / _GRID)), 0.0, 31.0)
    jy = jnp.clip(jnp.floor(y * (1.0 / _GRID)), 0.0, 31.0)
    c0 = jnp.clip(jx - 2.0, 0.0, 27.0)
    r0 = jnp.clip(jy - 2.0, 0.0, 27.0)
    lane = jax.lax.broadcasted_iota(jnp.int32, (n, 25), 1).astype(jnp.float32)
    dcol = lane - 5.0 * jnp.floor(lane * 0.2)
    drow = jnp.floor(lane * 0.2)
    cols = c0 + dcol
    rows = r0 + drow
    cx = cols * _GRID + 8.0
    cy = rows * _GRID + 8.0
    dx = x - cx
    dy = y - cy
    d2 = dx * dx + dy * dy
    idx = rows * 32.0 + cols
    ids = []
    for _ in range(4):
        m = jnp.min(d2, axis=1, keepdims=True)
        sel = jnp.min(jnp.where(d2 == m, idx, jnp.float32(4096.0)),
                      axis=1, keepdims=True)
        ids.append(sel)
        d2 = jnp.where(idx == sel, jnp.float32(_BIG), d2)
    return ids


def _onehot4(ids, lane_hw):
    acc = None
    for s in ids:
        plane = (lane_hw == s).astype(jnp.float32)
        acc = plane if acc is None else acc + plane
    return acc


def _loss_kernel(homo_ref, kp1_ref, wkp1_ref, kd_ref, d_ref, dt_ref, out_ref):
    i = pl.program_id(0)
    n = kd_ref.shape[1]
    hw = d_ref.shape[2]

    kd = kd_ref[0]
    D = d_ref[0]

    kx = kp1_ref[0, :, 0:1]
    ky = kp1_ref[0, :, 1:2]
    wx = wkp1_ref[0, :, 0:1]
    wy = wkp1_ref[0, :, 1:2]

    lane_hw = jax.lax.broadcasted_iota(jnp.int32, (n, hw), 1).astype(jnp.float32)

    kids = _nearest4(kx, ky)
    wids = _nearest4(wx, wy)
    K = _onehot4(kids, lane_hw)
    W = _onehot4(wids, lane_hw)

    h00 = homo_ref[i, 0]
    h01 = homo_ref[i, 1]
    h02 = homo_ref[i, 2]
    h10 = homo_ref[i, 3]
    h11 = homo_ref[i, 4]
    h12 = homo_ref[i, 5]
    h20 = homo_ref[i, 6]
    h21 = homo_ref[i, 7]
    h22 = homo_ref[i, 8]
    N = None
    for p in range(4):
        cidf = kids[p]
        row = jnp.floor(cidf * (1.0 / 32.0))
        col = cidf - 32.0 * row
        cx = col * _GRID + 8.0
        cy = row * _GRID + 8.0
        wz = h20 * cx + h21 * cy + h22
        px = (h00 * cx + h01 * cy + h02) / (wz + 1e-8)
        py = (h10 * cx + h11 * cy + h12) / (wz + 1e-8)
        cnt = _onehot4(_nearest4(px, py), lane_hw)
        N = cnt if N is None else N + cnt

    bx = wx * (1.0 / _GRID) - 0.5
    by = wy * (1.0 / _GRID) - 0.5
    x0 = jnp.floor(bx)
    y0 = jnp.floor(by)
    fx = bx - x0
    fy = by - y0
    x0c = jnp.clip(x0, 0.0, 31.0)
    x1c = jnp.clip(x0 + 1.0, 0.0, 31.0)
    y0c = jnp.clip(y0, 0.0, 31.0)
    y1c = jnp.clip(y0 + 1.0, 0.0, 31.0)
    A = ((lane_hw == y0c * 32.0 + x0c).astype(jnp.float32) * ((1 - fy) * (1 - fx))
         + (lane_hw == y0c * 32.0 + x1c).astype(jnp.float32) * ((1 - fy) * fx)
         + (lane_hw == y1c * 32.0 + x0c).astype(jnp.float32) * (fy * (1 - fx))
         + (lane_hw == y1c * 32.0 + x1c).astype(jnp.float32) * (fy * fx))
    nt = (((1,), (1,)), ((), ()))
    wd = jax.lax.dot_general(A, dt_ref[0], (((1,), (0,)), ((), ())),
                             preferred_element_type=jnp.float32)
    wd = wd * jax.lax.rsqrt(jnp.sum(wd * wd, axis=1, keepdims=True) + 1e-12)

    pos = 2.0 - 2.0 * jnp.sum(kd * wd, axis=1, keepdims=True)

    S = jax.lax.dot_general(kd, D, (((1,), (0,)), ((), ())),
                            preferred_element_type=jnp.float32)
    X = 2.0 - 2.0 * S + 5.0 * N
    fos_sum = jnp.float32(0.0)
    hwf = jnp.float32(hw)
    for _ in range(_NUM_NEG):
        m = jnp.min(X, axis=1, keepdims=True)
        sel = jnp.min(jnp.where(X == m, lane_hw, hwf), axis=1, keepdims=True)
        X = jnp.where(lane_hw == sel, jnp.float32(_BIG), X)
        t = jnp.maximum(pos - m + _MARGIN, 0.0)
        fos_sum = fos_sum + jnp.sum(t * t)

    Km = jax.lax.dot_general(K, K, nt, preferred_element_type=jnp.float32)
    Wm = jax.lax.dot_general(N, W, nt, preferred_element_type=jnp.float32)
    kraw = 2.0 - 2.0 * jax.lax.dot_general(kd, kd, nt,
                                           preferred_element_type=jnp.float32)
    wraw = 2.0 - 2.0 * jax.lax.dot_general(wd, wd, nt,
                                           preferred_element_type=jnp.float32)
    Xa = kraw + 5.0 * Km
    Xb = wraw + 5.0 * Wm
    lane_n = jax.lax.broadcasted_iota(jnp.int32, (n, n), 1).astype(jnp.float32)
    nf = jnp.float32(n)
    sacc = jnp.zeros((n, 1), jnp.float32)
    for _ in range(_SOS_NEG):
        ma = jnp.min(Xa, axis=1, keepdims=True)
        sa = jnp.min(jnp.where(Xa == ma, lane_n, nf), axis=1, keepdims=True)
        ea = lane_n == sa
        va = jnp.sum(jnp.where(ea, kraw, 0.0), axis=1, keepdims=True)
        Xa = jnp.where(ea, jnp.float32(_BIG), Xa)
        mb = jnp.min(Xb, axis=1, keepdims=True)
        sb = jnp.min(jnp.where(Xb == mb, lane_n, nf), axis=1, keepdims=True)
        eb = lane_n == sb
        vb = jnp.sum(jnp.where(eb, wraw, 0.0), axis=1, keepdims=True)
        Xb = jnp.where(eb, jnp.float32(_BIG), Xb)
        d = va - vb
        sacc = sacc + d * d
    sos_sum = jnp.sum(jnp.sqrt(sacc + 1e-12))

    lane_o = jax.lax.broadcasted_iota(jnp.int32, (1, 128), 1)
    out_ref[0] = jnp.where(lane_o == 0, fos_sum,
                           jnp.where(lane_o == 1, sos_sum, 0.0))


@jax.jit
def kernel(kp1, w_kp1, kp1_desc, desc2, homo12):
    b, n, c = kp1_desc.shape
    h, w = desc2.shape[2], desc2.shape[3]
    hw = h * w
    D = desc2.reshape(b, c, hw)
    homo_flat = homo12.reshape(b, 9)

    grid_spec = pltpu.PrefetchScalarGridSpec(
        num_scalar_prefetch=1,
        grid=(b,),
        in_specs=[
            pl.BlockSpec((1, n, 2), lambda i, s: (i, 0, 0)),
            pl.BlockSpec((1, n, 2), lambda i, s: (i, 0, 0)),
            pl.BlockSpec((1, n, c), lambda i, s: (i, 0, 0)),
            pl.BlockSpec((1, c, hw), lambda i, s: (i, 0, 0)),
            pl.BlockSpec((1, hw, c), lambda i, s: (i, 0, 0)),
        ],
        out_specs=pl.BlockSpec((1, 1, 128), lambda i, s: (i, 0, 0)),
    )
    part = pl.pallas_call(
        _loss_kernel,
        grid_spec=grid_spec,
        out_shape=jax.ShapeDtypeStruct((b, 1, 128), jnp.float32),
    )(homo_flat, kp1, w_kp1, kp1_desc, D, jnp.transpose(D, (0, 2, 1)))
    fos = jnp.sum(part[:, 0, 0]) / (b * n * _NUM_NEG)
    sos = jnp.sum(part[:, 0, 1]) / (b * n)
    return fos + sos
